# baseline (speedup 1.0000x reference)
import jax
import jax.numpy as jnp
from jax.experimental import pallas as pl
from jax.experimental.pallas import tpu as pltpu

_B, _C, _H, _W = 128, 10, 128, 128


def _body(tc_ref, g_ref, out_ref):
    g = g_ref[0]
    tc = tc_ref[0, 0]
    s2 = jnp.sum(g, axis=1)
    sums = jnp.sum(s2, axis=1, keepdims=True)
    smax = jnp.max(sums)
    ci2 = jax.lax.broadcasted_iota(jnp.int32, (_C, 1), 0)
    bg = jnp.min(jnp.where(sums == smax, ci2, _C))
    ci = jax.lax.broadcasted_iota(jnp.int32, (_C, 1, 1), 0)
    bg_grid = jnp.sum(jnp.where(ci == bg, g, 0.0), axis=0)
    m3 = (bg_grid > 0.5)[None]
    res = jnp.where((ci == bg) & m3, 0.0, g)
    res = jnp.where((ci == tc) & m3, 1.0, res)
    out_ref[0] = res


def kernel(grid, target_color):
    tgt = jnp.asarray(target_color, jnp.int32).reshape(1, 1)
    return pl.pallas_call(
        _body,
        grid=(_B,),
        in_specs=[
            pl.BlockSpec(memory_space=pltpu.SMEM),
            pl.BlockSpec((1, _C, _H, _W), lambda b: (b, 0, 0, 0)),
        ],
        out_specs=pl.BlockSpec((1, _C, _H, _W), lambda b: (b, 0, 0, 0)),
        out_shape=jax.ShapeDtypeStruct((_B, _C, _H, _W), jnp.float32),
    )(tgt, grid)


# speedup vs baseline: 1.1832x; 1.1832x over previous
import jax
import jax.numpy as jnp
from jax.experimental import pallas as pl
from jax.experimental.pallas import tpu as pltpu

_B, _C, _H, _W = 128, 10, 128, 128


def _body(tc_ref, g_ref, out_ref):
    g = g_ref[0]
    tc = tc_ref[0, 0]
    s2 = jnp.sum(g, axis=1)
    sums = jnp.sum(s2, axis=1, keepdims=True)
    smax = jnp.max(sums)
    ci2 = jax.lax.broadcasted_iota(jnp.int32, (_C, 1), 0)
    bg = jnp.min(jnp.where(sums == smax, ci2, _C))
    out_ref[0] = g
    bgr = g_ref[0, bg]
    mask = bgr > 0.5
    out_ref[0, bg] = jnp.where(mask, 0.0, bgr)
    tr = g_ref[0, tc]
    out_ref[0, tc] = jnp.where(mask, 1.0, tr)


def kernel(grid, target_color):
    tgt = jnp.asarray(target_color, jnp.int32).reshape(1, 1)
    return pl.pallas_call(
        _body,
        grid=(_B,),
        in_specs=[
            pl.BlockSpec(memory_space=pltpu.SMEM),
            pl.BlockSpec((1, _C, _H, _W), lambda b: (b, 0, 0, 0)),
        ],
        out_specs=pl.BlockSpec((1, _C, _H, _W), lambda b: (b, 0, 0, 0)),
        out_shape=jax.ShapeDtypeStruct((_B, _C, _H, _W), jnp.float32),
    )(tgt, grid)


# speedup vs baseline: 2.4196x; 2.0449x over previous
import jax
import jax.numpy as jnp
from jax.experimental import pallas as pl
from jax.experimental.pallas import tpu as pltpu

_B, _C, _H, _W = 128, 10, 128, 128


_NB = 8


def _body(tc_ref, g_ref, out_ref):
    tc = tc_ref[0, 0]
    out_ref[...] = g_ref[...]
    for i in range(_NB):
        g = g_ref[i]
        s2 = jnp.sum(g, axis=1)
        sums = jnp.sum(s2, axis=1, keepdims=True)
        smax = jnp.max(sums)
        ci2 = jax.lax.broadcasted_iota(jnp.int32, (_C, 1), 0)
        bg = jnp.min(jnp.where(sums == smax, ci2, _C))
        bgr = g_ref[i, bg]
        mask = bgr > 0.5
        out_ref[i, bg] = jnp.where(mask, 0.0, bgr)
        tr = g_ref[i, tc]
        out_ref[i, tc] = jnp.where(mask, 1.0, tr)


def kernel(grid, target_color):
    tgt = jnp.asarray(target_color, jnp.int32).reshape(1, 1)
    return pl.pallas_call(
        _body,
        grid=(_B // _NB,),
        in_specs=[
            pl.BlockSpec(memory_space=pltpu.SMEM),
            pl.BlockSpec((_NB, _C, _H, _W), lambda b: (b, 0, 0, 0)),
        ],
        out_specs=pl.BlockSpec((_NB, _C, _H, _W), lambda b: (b, 0, 0, 0)),
        out_shape=jax.ShapeDtypeStruct((_B, _C, _H, _W), jnp.float32),
    )(tgt, grid)


# speedup vs baseline: 2.5020x; 1.0341x over previous
import jax
import jax.numpy as jnp
from jax.experimental import pallas as pl
from jax.experimental.pallas import tpu as pltpu

_B, _C, _H, _W = 128, 10, 128, 128


_NB = 16


def _body(tc_ref, g_ref, out_ref):
    tc = tc_ref[0, 0]
    out_ref[...] = g_ref[...]
    for i in range(_NB):
        g = g_ref[i]
        s2 = jnp.sum(g, axis=1)
        sums = jnp.sum(s2, axis=1, keepdims=True)
        smax = jnp.max(sums)
        ci2 = jax.lax.broadcasted_iota(jnp.int32, (_C, 1), 0)
        bg = jnp.min(jnp.where(sums == smax, ci2, _C))
        bgr = g_ref[i, bg]
        mask = bgr > 0.5
        out_ref[i, bg] = jnp.where(mask, 0.0, bgr)
        tr = g_ref[i, tc]
        out_ref[i, tc] = jnp.where(mask, 1.0, tr)


def kernel(grid, target_color):
    tgt = jnp.asarray(target_color, jnp.int32).reshape(1, 1)
    return pl.pallas_call(
        _body,
        grid=(_B // _NB,),
        in_specs=[
            pl.BlockSpec(memory_space=pltpu.SMEM),
            pl.BlockSpec((_NB, _C, _H, _W), lambda b: (b, 0, 0, 0)),
        ],
        out_specs=pl.BlockSpec((_NB, _C, _H, _W), lambda b: (b, 0, 0, 0)),
        out_shape=jax.ShapeDtypeStruct((_B, _C, _H, _W), jnp.float32),
    )(tgt, grid)
